# baseline (speedup 1.0000x reference)
import functools

import jax
import jax.numpy as jnp
from jax import lax
from jax.experimental import pallas as pl
from jax.experimental.pallas import tpu as pltpu
from jax.experimental.pallas import tpu_sc as plsc

_SUBCORES = 16
_NUM_CORES = 2

BLK = 128
SUPER = 8
BN = 512
HTW = 48
G = 64


def _cdiv(a, b):
  return (a + b - 1) // b


def _edge_pass_body(hc, npad, nblk,
                    ht_hbm, alt_hbm, e2_hbm, zu_hbm, zs_hbm,
                    u2_hbm, s2_hbm,
                    ebuf, dbuf, d8buf, is2, id2, mbuf, gbuf, abuf,
                    cbuf, csbuf, wbuf, acc_u, acc_s, semg, semsc):
  c = lax.axis_index("c")
  t = lax.axis_index("s")
  np8 = npad // 8
  rpt_u = npad // _SUBCORES
  rpt_s = np8 // _SUBCORES

  pltpu.sync_copy(zu_hbm, acc_u.at[pl.ds(t * rpt_u, rpt_u)])
  pltpu.sync_copy(zs_hbm, acc_s.at[pl.ds(t * rpt_s, rpt_s)])
  plsc.subcore_barrier()

  sbw = SUPER * BLK
  base0 = t * (nblk * BLK)
  off = c * npad
  iota = lax.iota(jnp.int32, 16)

  def outer_body(js, carry):
    pltpu.sync_copy(e2_hbm.at[:, pl.ds(base0 + js * sbw, sbw)], ebuf)

    def inner_body(sb, carry2):
      @pl.when(js * SUPER + sb > 0)
      def _():
        pltpu.make_async_copy(cbuf, acc_u.at[dbuf], semsc).wait()
        pltpu.make_async_copy(csbuf, acc_s.at[d8buf], semsc).wait()

      eb = sb * BLK
      for g in range(BLK // 16):
        sl = pl.ds(g * 16, 16)
        s16 = ebuf[0, pl.ds(eb + g * 16, 16)]
        d16 = ebuf[1, pl.ds(eb + g * 16, 16)]
        is2[sl] = s16 + off
        id2[sl] = d16 + off
        dbuf[sl] = d16
        d8buf[sl] = lax.shift_right_logical(d16, 3)
        mbuf[sl] = lax.shift_left(jnp.bitwise_and(d16, 7), 1)
      cp1 = pltpu.async_copy(ht_hbm.at[is2], gbuf, semg)
      cp2 = pltpu.async_copy(alt_hbm.at[id2], abuf, semg)
      cp1.wait()
      cp2.wait()

      def grp_body(g, carry3):
        mv = mbuf[pl.ds(g * 16, 16)]
        for kk in range(16):
          k = g * 16 + kk
          sv = gbuf[k, pl.ds(32, 16)]
          dv = abuf[k, pl.ds(0, 16)]
          ev = sv + dv
          ev = jnp.maximum(ev, 0.2 * ev)
          wbuf[kk, pl.ds(0, 16)] = jnp.exp(ev)
        for kk in range(16):
          k = g * 16 + kk
          wv = wbuf[kk, pl.ds(0, 16)]
          w0 = wv[0]
          w1 = wv[1] if hc == 2 else w0
          cbuf[k, pl.ds(0, 16)] = gbuf[k, pl.ds(0, 16)] * w0
          cbuf[k, pl.ds(16, 16)] = gbuf[k, pl.ds(16, 16)] * w1
          m = mv[kk]
          ws = jnp.where(iota == m, w0, 0.0)
          if hc == 2:
            ws = jnp.where(iota == m + 1, w1, ws)
          csbuf[k, pl.ds(0, 16)] = ws
        return carry3

      lax.fori_loop(0, BLK // 16, grp_body, 0)
      pltpu.async_copy(cbuf, acc_u.at[dbuf], semsc, add=True)
      pltpu.async_copy(csbuf, acc_s.at[d8buf], semsc, add=True)
      return carry2

    lax.fori_loop(0, SUPER, inner_body, 0)
    return carry

  lax.fori_loop(0, nblk // SUPER, outer_body, 0)
  pltpu.make_async_copy(cbuf, acc_u.at[dbuf], semsc).wait()
  pltpu.make_async_copy(csbuf, acc_s.at[d8buf], semsc).wait()
  plsc.subcore_barrier()
  pltpu.sync_copy(acc_u.at[pl.ds(t * rpt_u, rpt_u)],
                  u2_hbm.at[c, pl.ds(t * rpt_u, rpt_u)])
  pltpu.sync_copy(acc_s.at[pl.ds(t * rpt_s, rpt_s)],
                  s2_hbm.at[c, pl.ds(t * rpt_s, rpt_s)])

def _edge_pass(hc, npad, ep, ht_flat, alt_flat, e2):
  nsub = _SUBCORES
  np8 = npad // 8
  nblk = ep // (nsub * BLK)
  zu = jnp.zeros((npad // nsub, 32), jnp.float32)
  zs = jnp.zeros((np8 // nsub, 16), jnp.float32)
  mesh = plsc.VectorSubcoreMesh(core_axis_name="c", subcore_axis_name="s",
                                num_cores=_NUM_CORES, num_subcores=_SUBCORES)
  body = functools.partial(_edge_pass_body, hc, npad, nblk)
  fn = pl.kernel(
      body,
      out_type=[jax.ShapeDtypeStruct((2, npad, 32), jnp.float32),
                jax.ShapeDtypeStruct((2, np8, 16), jnp.float32)],
      mesh=mesh,
      scratch_types=[
          pltpu.VMEM((2, SUPER * BLK), jnp.int32),
          pltpu.VMEM((BLK,), jnp.int32),
          pltpu.VMEM((BLK,), jnp.int32),
          pltpu.VMEM((BLK,), jnp.int32),
          pltpu.VMEM((BLK,), jnp.int32),
          pltpu.VMEM((BLK,), jnp.int32),
          pltpu.VMEM((BLK, HTW), jnp.float32),
          pltpu.VMEM((BLK, 16), jnp.float32),
          pltpu.VMEM((BLK, 32), jnp.float32),
          pltpu.VMEM((BLK, 16), jnp.float32),
          pltpu.VMEM((16, 16), jnp.float32),
          pltpu.VMEM_SHARED((npad, 32), jnp.float32),
          pltpu.VMEM_SHARED((np8, 16), jnp.float32),
          pltpu.SemaphoreType.DMA,
          pltpu.SemaphoreType.DMA,
      ],
      compiler_params=pltpu.CompilerParams(use_tc_tiling_on_sc=False),
  )
  return fn(ht_flat, alt_flat, e2, zu, zs)


def _pack_tables(h, als, ald, heads):
  bn = h.shape[0]
  if heads == 4:
    zt = jnp.zeros((bn, HTW - 34), jnp.float32)
    za = jnp.zeros((bn, 14), jnp.float32)
    r0 = jnp.concatenate([h[:, :32], als[:, 0:2], zt], axis=1)
    r1 = jnp.concatenate([h[:, 32:], als[:, 2:4], zt], axis=1)
    a0 = jnp.concatenate([ald[:, 0:2], za], axis=1)
    a1 = jnp.concatenate([ald[:, 2:4], za], axis=1)
  else:
    zt = jnp.zeros((bn, HTW - 33), jnp.float32)
    za = jnp.zeros((bn, 15), jnp.float32)
    r0 = jnp.concatenate([h[:, :32], als, zt], axis=1)
    r1 = jnp.concatenate([h[:, 32:], als, zt], axis=1)
    a0 = jnp.concatenate([ald, za], axis=1)
    a1 = a0
  return jnp.stack([r0, r1], axis=0), jnp.stack([a0, a1], axis=0)


def _dense_tables_body(heads, x_ref, w_ref, as_ref, ad_ref, ht_ref, alt_ref):
  xb = x_ref[...]
  h = jnp.dot(xb, w_ref[...], preferred_element_type=jnp.float32)
  als = jnp.dot(h, as_ref[...], preferred_element_type=jnp.float32)
  ald = jnp.dot(h, ad_ref[...], preferred_element_type=jnp.float32)
  ht, alt = _pack_tables(h, als, ald, heads)
  ht_ref[...] = ht
  alt_ref[...] = alt


def _combine(u_ref, s_ref, heads_prev, gg, be, b):
  u = jnp.concatenate([u_ref[0], u_ref[1]], axis=1)
  bn = u.shape[0]
  if heads_prev == 4:
    s64 = jnp.concatenate(
        [jnp.broadcast_to(s_ref[:, hh:hh + 1], (bn, 16)) for hh in range(4)],
        axis=1)
  else:
    s64 = jnp.broadcast_to(s_ref[:, 0:1], (bn, 64))
  out = u / (s64 + 1e-16) + b
  return jnp.maximum(gg * out + be, 0.0)


def _combine_dense_body(heads_prev, heads, u_ref, s_ref, g_ref, be_ref, b_ref,
                        w_ref, as_ref, ad_ref, ht_ref, alt_ref):
  act = _combine(u_ref, s_ref, heads_prev, g_ref[...], be_ref[...], b_ref[...])
  h = jnp.dot(act, w_ref[...], preferred_element_type=jnp.float32)
  als = jnp.dot(h, as_ref[...], preferred_element_type=jnp.float32)
  ald = jnp.dot(h, ad_ref[...], preferred_element_type=jnp.float32)
  ht, alt = _pack_tables(h, als, ald, heads)
  ht_ref[...] = ht
  alt_ref[...] = alt


def _final_body(nb, u_ref, s_ref, batch_ref, g_ref, be_ref, b_ref,
                fc1w_ref, fc1b_ref, fc2w_ref, fc2b_ref, out_ref, acc):
  i = pl.program_id(0)

  @pl.when(i == 0)
  def _():
    acc[...] = jnp.zeros_like(acc)

  h = _combine(u_ref, s_ref, 1, g_ref[...], be_ref[...], b_ref[...])
  bn = h.shape[0]
  bb = batch_ref[0, 0, :]
  oh = (lax.broadcasted_iota(jnp.int32, (G, bn), 0) == bb[None, :]).astype(
      jnp.float32)
  hplus = jnp.concatenate([h, jnp.ones((bn, 8), jnp.float32)], axis=1)
  acc[...] += jnp.dot(oh, hplus, preferred_element_type=jnp.float32)

  @pl.when(i == nb - 1)
  def _():
    pooled = acc[:, 0:64] / jnp.maximum(acc[:, 64:65], 1.0)
    z = jnp.maximum(
        jnp.dot(pooled, fc1w_ref[...], preferred_element_type=jnp.float32)
        + fc1b_ref[...], 0.0)
    out_ref[...] = (jnp.dot(z, fc2w_ref[...], preferred_element_type=jnp.float32)
                    + fc2b_ref[...])


def _table_out_shapes(npad):
  return (jax.ShapeDtypeStruct((2, npad, HTW), jnp.float32),
          jax.ShapeDtypeStruct((2, npad, 16), jnp.float32))


def _table_out_specs():
  return (pl.BlockSpec((2, BN, HTW), lambda i: (0, i, 0)),
          pl.BlockSpec((2, BN, 16), lambda i: (0, i, 0)))


def _full_spec(shape):
  nd = len(shape)
  return pl.BlockSpec(shape, lambda i: (0,) * nd)


def _dense_tables(heads, npad, x_pad, w, a_s, a_d):
  nb = npad // BN
  cin = x_pad.shape[1]
  return pl.pallas_call(
      functools.partial(_dense_tables_body, heads),
      grid=(nb,),
      in_specs=[
          pl.BlockSpec((BN, cin), lambda i: (i, 0)),
          _full_spec(w.shape),
          _full_spec(a_s.shape),
          _full_spec(a_d.shape),
      ],
      out_specs=list(_table_out_specs()),
      out_shape=list(_table_out_shapes(npad)),
  )(x_pad, w, a_s, a_d)


def _combine_dense(heads_prev, heads, npad, u2, s4, gg, be, b, w, a_s, a_d):
  nb = npad // BN
  return pl.pallas_call(
      functools.partial(_combine_dense_body, heads_prev, heads),
      grid=(nb,),
      in_specs=[
          pl.BlockSpec((2, BN, 32), lambda i: (0, i, 0)),
          pl.BlockSpec((BN, 4), lambda i: (i, 0)),
          _full_spec((1, 64)), _full_spec((1, 64)), _full_spec((1, 64)),
          _full_spec(w.shape),
          _full_spec(a_s.shape),
          _full_spec(a_d.shape),
      ],
      out_specs=list(_table_out_specs()),
      out_shape=list(_table_out_shapes(npad)),
  )(u2, s4, gg.reshape(1, 64), be.reshape(1, 64), b.reshape(1, 64), w, a_s, a_d)


def _final(npad, u2, s4, batch_r, gg, be, b, fc1_w, fc1_b, fc2_w, fc2_b):
  nb = npad // BN
  return pl.pallas_call(
      functools.partial(_final_body, nb),
      grid=(nb,),
      in_specs=[
          pl.BlockSpec((2, BN, 32), lambda i: (0, i, 0)),
          pl.BlockSpec((BN, 4), lambda i: (i, 0)),
          pl.BlockSpec((1, 1, BN), lambda i: (i, 0, 0)),
          _full_spec((1, 64)), _full_spec((1, 64)), _full_spec((1, 64)),
          _full_spec((64, 64)), _full_spec((1, 64)),
          _full_spec((64, 1)), _full_spec((1, 1)),
      ],
      out_specs=pl.BlockSpec((G, 1), lambda i: (0, 0)),
      out_shape=jax.ShapeDtypeStruct((G, 1), jnp.float32),
      scratch_shapes=[pltpu.VMEM((G, 72), jnp.float32)],
  )(u2, s4, batch_r, gg.reshape(1, 64), be.reshape(1, 64), b.reshape(1, 64),
    fc1_w, fc1_b.reshape(1, 64), fc2_w, fc2_b.reshape(1, 1))


def _attn_mat(a, heads):
  if heads == 1:
    return a.T.astype(jnp.float32)
  cols = [a[h][:, None] for h in range(heads)]
  return jax.scipy.linalg.block_diag(*cols).astype(jnp.float32)


def _s_pairs(s2, npad):
  sp = s2.reshape(2, npad, 2)
  return jnp.concatenate([sp[0], sp[1]], axis=1)


def kernel(x, edge_index, batch, W1, a_src1, a_dst1, b1, g1, be1,
           W2, a_src2, a_dst2, b2, g2, be2, W3, a_src3, a_dst3, b3, g3, be3,
           fc1_w, fc1_b, fc2_w, fc2_b):
  n = x.shape[0]
  e = edge_index.shape[1]
  npad = _cdiv(n, BN) * BN
  etot = e + n
  ep = _cdiv(etot, _SUBCORES * BLK * SUPER) * (_SUBCORES * BLK * SUPER)

  loops = jnp.arange(n, dtype=jnp.int32)
  padv = jnp.full((ep - etot,), n, jnp.int32)
  srcv = jnp.concatenate([edge_index[0], loops, padv])
  dstv = jnp.concatenate([edge_index[1], loops, padv])
  e2 = jnp.stack([srcv, dstv])

  x_pad = jnp.pad(x, ((0, npad - n), (0, 0)))
  batch_r = jnp.pad(batch, (0, npad - n), constant_values=G).reshape(
      npad // BN, 1, BN)

  ht, alt = _dense_tables(4, npad, x_pad, W1, _attn_mat(a_src1, 4),
                          _attn_mat(a_dst1, 4))
  u2, s2 = _edge_pass(2, npad, ep, ht.reshape(2 * npad, HTW),
                      alt.reshape(2 * npad, 16), e2)
  ht, alt = _combine_dense(4, 4, npad, u2, _s_pairs(s2, npad), g1, be1, b1,
                           W2, _attn_mat(a_src2, 4), _attn_mat(a_dst2, 4))
  u2, s2 = _edge_pass(2, npad, ep, ht.reshape(2 * npad, HTW),
                      alt.reshape(2 * npad, 16), e2)
  ht, alt = _combine_dense(4, 1, npad, u2, _s_pairs(s2, npad), g2, be2, b2,
                           W3, _attn_mat(a_src3, 1), _attn_mat(a_dst3, 1))
  u2, s2 = _edge_pass(1, npad, ep, ht.reshape(2 * npad, HTW),
                      alt.reshape(2 * npad, 16), e2)
  return _final(npad, u2, _s_pairs(s2, npad), batch_r, g3, be3, b3,
                fc1_w, fc1_b, fc2_w, fc2_b)


# speedup vs baseline: 1.0895x; 1.0895x over previous
import functools

import jax
import jax.numpy as jnp
from jax import lax
from jax.experimental import pallas as pl
from jax.experimental.pallas import tpu as pltpu
from jax.experimental.pallas import tpu_sc as plsc

_SUBCORES = 16
_NUM_CORES = 2

BLK = 128
SUPER = 8
BN = 512
HTW = 48
G = 64


def _cdiv(a, b):
  return (a + b - 1) // b


def _edge_pass_body(hc, npad, nblk,
                    ht_hbm, alt_hbm, e2_hbm, zu_hbm, zs_hbm,
                    u2_hbm, s2_hbm,
                    ebuf, dbuf, d8buf, is2, id2, mbuf, gbuf, abuf,
                    cbuf, csbuf, acc_u, acc_s, semg, semsc):
  c = lax.axis_index("c")
  t = lax.axis_index("s")
  np8 = npad // 8
  rpt_u = npad // _SUBCORES
  rpt_s = np8 // _SUBCORES

  pltpu.sync_copy(zu_hbm, acc_u.at[pl.ds(t * rpt_u, rpt_u)])
  pltpu.sync_copy(zs_hbm, acc_s.at[pl.ds(t * rpt_s, rpt_s)])
  plsc.subcore_barrier()

  sbw = SUPER * BLK
  base0 = t * (nblk * BLK)
  off = c * npad
  iota = lax.iota(jnp.int32, 16)

  def outer_body(js, carry):
    pltpu.sync_copy(e2_hbm.at[:, pl.ds(base0 + js * sbw, sbw)], ebuf)

    def inner_body(sb, carry2):
      @pl.when(js * SUPER + sb > 0)
      def _():
        pltpu.make_async_copy(cbuf, acc_u.at[dbuf], semsc).wait()
        pltpu.make_async_copy(csbuf, acc_s.at[d8buf], semsc).wait()

      eb = sb * BLK
      for g in range(BLK // 16):
        sl = pl.ds(g * 16, 16)
        s16 = ebuf[0, pl.ds(eb + g * 16, 16)]
        d16 = ebuf[1, pl.ds(eb + g * 16, 16)]
        is2[sl] = s16 + off
        id2[sl] = d16 + off
        dbuf[sl] = d16
        d8buf[sl] = lax.shift_right_logical(d16, 3)
        mbuf[sl] = lax.shift_left(jnp.bitwise_and(d16, 7), 1)
      cp1 = pltpu.async_copy(ht_hbm.at[is2], gbuf, semg)
      cp2 = pltpu.async_copy(alt_hbm.at[id2], abuf, semg)
      cp1.wait()
      cp2.wait()

      def grp_body(g, carry3):
        mv = mbuf[pl.ds(g * 16, 16)]
        for kk in range(16):
          k = g * 16 + kk
          sv = gbuf[k, pl.ds(32, 16)]
          dv = abuf[k, pl.ds(0, 16)]
          ev = sv + dv
          ev = jnp.maximum(ev, 0.2 * ev)
          wv = jnp.exp(ev)
          w0 = wv[0]
          w1 = wv[1] if hc == 2 else w0
          cbuf[k, pl.ds(0, 16)] = gbuf[k, pl.ds(0, 16)] * w0
          cbuf[k, pl.ds(16, 16)] = gbuf[k, pl.ds(16, 16)] * w1
          m = mv[kk]
          ws = jnp.where(iota == m, w0, 0.0)
          if hc == 2:
            ws = jnp.where(iota == m + 1, w1, ws)
          csbuf[k, pl.ds(0, 16)] = ws
        return carry3

      lax.fori_loop(0, BLK // 16, grp_body, 0)
      pltpu.async_copy(cbuf, acc_u.at[dbuf], semsc, add=True)
      pltpu.async_copy(csbuf, acc_s.at[d8buf], semsc, add=True)
      return carry2

    lax.fori_loop(0, SUPER, inner_body, 0)
    return carry

  lax.fori_loop(0, nblk // SUPER, outer_body, 0)
  pltpu.make_async_copy(cbuf, acc_u.at[dbuf], semsc).wait()
  pltpu.make_async_copy(csbuf, acc_s.at[d8buf], semsc).wait()
  plsc.subcore_barrier()
  pltpu.sync_copy(acc_u.at[pl.ds(t * rpt_u, rpt_u)],
                  u2_hbm.at[c, pl.ds(t * rpt_u, rpt_u)])
  pltpu.sync_copy(acc_s.at[pl.ds(t * rpt_s, rpt_s)],
                  s2_hbm.at[c, pl.ds(t * rpt_s, rpt_s)])

def _edge_pass(hc, npad, ep, ht_flat, alt_flat, e2):
  nsub = _SUBCORES
  np8 = npad // 8
  nblk = ep // (nsub * BLK)
  zu = jnp.zeros((npad // nsub, 32), jnp.float32)
  zs = jnp.zeros((np8 // nsub, 16), jnp.float32)
  mesh = plsc.VectorSubcoreMesh(core_axis_name="c", subcore_axis_name="s",
                                num_cores=_NUM_CORES, num_subcores=_SUBCORES)
  body = functools.partial(_edge_pass_body, hc, npad, nblk)
  fn = pl.kernel(
      body,
      out_type=[jax.ShapeDtypeStruct((2, npad, 32), jnp.float32),
                jax.ShapeDtypeStruct((2, np8, 16), jnp.float32)],
      mesh=mesh,
      scratch_types=[
          pltpu.VMEM((2, SUPER * BLK), jnp.int32),
          pltpu.VMEM((BLK,), jnp.int32),
          pltpu.VMEM((BLK,), jnp.int32),
          pltpu.VMEM((BLK,), jnp.int32),
          pltpu.VMEM((BLK,), jnp.int32),
          pltpu.VMEM((BLK,), jnp.int32),
          pltpu.VMEM((BLK, HTW), jnp.float32),
          pltpu.VMEM((BLK, 16), jnp.float32),
          pltpu.VMEM((BLK, 32), jnp.float32),
          pltpu.VMEM((BLK, 16), jnp.float32),
          pltpu.VMEM_SHARED((npad, 32), jnp.float32),
          pltpu.VMEM_SHARED((np8, 16), jnp.float32),
          pltpu.SemaphoreType.DMA,
          pltpu.SemaphoreType.DMA,
      ],
      compiler_params=pltpu.CompilerParams(use_tc_tiling_on_sc=False),
  )
  return fn(ht_flat, alt_flat, e2, zu, zs)


def _pack_tables(h, als, ald, heads):
  bn = h.shape[0]
  if heads == 4:
    zt = jnp.zeros((bn, HTW - 34), jnp.float32)
    za = jnp.zeros((bn, 14), jnp.float32)
    r0 = jnp.concatenate([h[:, :32], als[:, 0:2], zt], axis=1)
    r1 = jnp.concatenate([h[:, 32:], als[:, 2:4], zt], axis=1)
    a0 = jnp.concatenate([ald[:, 0:2], za], axis=1)
    a1 = jnp.concatenate([ald[:, 2:4], za], axis=1)
  else:
    zt = jnp.zeros((bn, HTW - 33), jnp.float32)
    za = jnp.zeros((bn, 15), jnp.float32)
    r0 = jnp.concatenate([h[:, :32], als, zt], axis=1)
    r1 = jnp.concatenate([h[:, 32:], als, zt], axis=1)
    a0 = jnp.concatenate([ald, za], axis=1)
    a1 = a0
  return jnp.stack([r0, r1], axis=0), jnp.stack([a0, a1], axis=0)


def _dense_tables_body(heads, x_ref, w_ref, as_ref, ad_ref, ht_ref, alt_ref):
  xb = x_ref[...]
  h = jnp.dot(xb, w_ref[...], preferred_element_type=jnp.float32)
  als = jnp.dot(h, as_ref[...], preferred_element_type=jnp.float32)
  ald = jnp.dot(h, ad_ref[...], preferred_element_type=jnp.float32)
  ht, alt = _pack_tables(h, als, ald, heads)
  ht_ref[...] = ht
  alt_ref[...] = alt


def _combine(u_ref, s_ref, heads_prev, gg, be, b):
  u = jnp.concatenate([u_ref[0], u_ref[1]], axis=1)
  bn = u.shape[0]
  if heads_prev == 4:
    s64 = jnp.concatenate(
        [jnp.broadcast_to(s_ref[:, hh:hh + 1], (bn, 16)) for hh in range(4)],
        axis=1)
  else:
    s64 = jnp.broadcast_to(s_ref[:, 0:1], (bn, 64))
  out = u / (s64 + 1e-16) + b
  return jnp.maximum(gg * out + be, 0.0)


def _combine_dense_body(heads_prev, heads, u_ref, s_ref, g_ref, be_ref, b_ref,
                        w_ref, as_ref, ad_ref, ht_ref, alt_ref):
  act = _combine(u_ref, s_ref, heads_prev, g_ref[...], be_ref[...], b_ref[...])
  h = jnp.dot(act, w_ref[...], preferred_element_type=jnp.float32)
  als = jnp.dot(h, as_ref[...], preferred_element_type=jnp.float32)
  ald = jnp.dot(h, ad_ref[...], preferred_element_type=jnp.float32)
  ht, alt = _pack_tables(h, als, ald, heads)
  ht_ref[...] = ht
  alt_ref[...] = alt


def _final_body(nb, u_ref, s_ref, batch_ref, g_ref, be_ref, b_ref,
                fc1w_ref, fc1b_ref, fc2w_ref, fc2b_ref, out_ref, acc):
  i = pl.program_id(0)

  @pl.when(i == 0)
  def _():
    acc[...] = jnp.zeros_like(acc)

  h = _combine(u_ref, s_ref, 1, g_ref[...], be_ref[...], b_ref[...])
  bn = h.shape[0]
  bb = batch_ref[0, 0, :]
  oh = (lax.broadcasted_iota(jnp.int32, (G, bn), 0) == bb[None, :]).astype(
      jnp.float32)
  hplus = jnp.concatenate([h, jnp.ones((bn, 8), jnp.float32)], axis=1)
  acc[...] += jnp.dot(oh, hplus, preferred_element_type=jnp.float32)

  @pl.when(i == nb - 1)
  def _():
    pooled = acc[:, 0:64] / jnp.maximum(acc[:, 64:65], 1.0)
    z = jnp.maximum(
        jnp.dot(pooled, fc1w_ref[...], preferred_element_type=jnp.float32)
        + fc1b_ref[...], 0.0)
    out_ref[...] = (jnp.dot(z, fc2w_ref[...], preferred_element_type=jnp.float32)
                    + fc2b_ref[...])


def _table_out_shapes(npad):
  return (jax.ShapeDtypeStruct((2, npad, HTW), jnp.float32),
          jax.ShapeDtypeStruct((2, npad, 16), jnp.float32))


def _table_out_specs():
  return (pl.BlockSpec((2, BN, HTW), lambda i: (0, i, 0)),
          pl.BlockSpec((2, BN, 16), lambda i: (0, i, 0)))


def _full_spec(shape):
  nd = len(shape)
  return pl.BlockSpec(shape, lambda i: (0,) * nd)


def _dense_tables(heads, npad, x_pad, w, a_s, a_d):
  nb = npad // BN
  cin = x_pad.shape[1]
  return pl.pallas_call(
      functools.partial(_dense_tables_body, heads),
      grid=(nb,),
      in_specs=[
          pl.BlockSpec((BN, cin), lambda i: (i, 0)),
          _full_spec(w.shape),
          _full_spec(a_s.shape),
          _full_spec(a_d.shape),
      ],
      out_specs=list(_table_out_specs()),
      out_shape=list(_table_out_shapes(npad)),
  )(x_pad, w, a_s, a_d)


def _combine_dense(heads_prev, heads, npad, u2, s4, gg, be, b, w, a_s, a_d):
  nb = npad // BN
  return pl.pallas_call(
      functools.partial(_combine_dense_body, heads_prev, heads),
      grid=(nb,),
      in_specs=[
          pl.BlockSpec((2, BN, 32), lambda i: (0, i, 0)),
          pl.BlockSpec((BN, 4), lambda i: (i, 0)),
          _full_spec((1, 64)), _full_spec((1, 64)), _full_spec((1, 64)),
          _full_spec(w.shape),
          _full_spec(a_s.shape),
          _full_spec(a_d.shape),
      ],
      out_specs=list(_table_out_specs()),
      out_shape=list(_table_out_shapes(npad)),
  )(u2, s4, gg.reshape(1, 64), be.reshape(1, 64), b.reshape(1, 64), w, a_s, a_d)


def _final(npad, u2, s4, batch_r, gg, be, b, fc1_w, fc1_b, fc2_w, fc2_b):
  nb = npad // BN
  return pl.pallas_call(
      functools.partial(_final_body, nb),
      grid=(nb,),
      in_specs=[
          pl.BlockSpec((2, BN, 32), lambda i: (0, i, 0)),
          pl.BlockSpec((BN, 4), lambda i: (i, 0)),
          pl.BlockSpec((1, 1, BN), lambda i: (i, 0, 0)),
          _full_spec((1, 64)), _full_spec((1, 64)), _full_spec((1, 64)),
          _full_spec((64, 64)), _full_spec((1, 64)),
          _full_spec((64, 1)), _full_spec((1, 1)),
      ],
      out_specs=pl.BlockSpec((G, 1), lambda i: (0, 0)),
      out_shape=jax.ShapeDtypeStruct((G, 1), jnp.float32),
      scratch_shapes=[pltpu.VMEM((G, 72), jnp.float32)],
  )(u2, s4, batch_r, gg.reshape(1, 64), be.reshape(1, 64), b.reshape(1, 64),
    fc1_w, fc1_b.reshape(1, 64), fc2_w, fc2_b.reshape(1, 1))


def _attn_mat(a, heads):
  if heads == 1:
    return a.T.astype(jnp.float32)
  cols = [a[h][:, None] for h in range(heads)]
  return jax.scipy.linalg.block_diag(*cols).astype(jnp.float32)


def _s_pairs(s2, npad):
  sp = s2.reshape(2, npad, 2)
  return jnp.concatenate([sp[0], sp[1]], axis=1)


def kernel(x, edge_index, batch, W1, a_src1, a_dst1, b1, g1, be1,
           W2, a_src2, a_dst2, b2, g2, be2, W3, a_src3, a_dst3, b3, g3, be3,
           fc1_w, fc1_b, fc2_w, fc2_b):
  n = x.shape[0]
  e = edge_index.shape[1]
  npad = _cdiv(n, BN) * BN
  etot = e + n
  ep = _cdiv(etot, _SUBCORES * BLK * SUPER) * (_SUBCORES * BLK * SUPER)

  loops = jnp.arange(n, dtype=jnp.int32)
  padv = jnp.full((ep - etot,), n, jnp.int32)
  srcv = jnp.concatenate([edge_index[0], loops, padv])
  dstv = jnp.concatenate([edge_index[1], loops, padv])
  e2 = jnp.stack([srcv, dstv])

  x_pad = jnp.pad(x, ((0, npad - n), (0, 0)))
  batch_r = jnp.pad(batch, (0, npad - n), constant_values=G).reshape(
      npad // BN, 1, BN)

  ht, alt = _dense_tables(4, npad, x_pad, W1, _attn_mat(a_src1, 4),
                          _attn_mat(a_dst1, 4))
  u2, s2 = _edge_pass(2, npad, ep, ht.reshape(2 * npad, HTW),
                      alt.reshape(2 * npad, 16), e2)
  ht, alt = _combine_dense(4, 4, npad, u2, _s_pairs(s2, npad), g1, be1, b1,
                           W2, _attn_mat(a_src2, 4), _attn_mat(a_dst2, 4))
  u2, s2 = _edge_pass(2, npad, ep, ht.reshape(2 * npad, HTW),
                      alt.reshape(2 * npad, 16), e2)
  ht, alt = _combine_dense(4, 1, npad, u2, _s_pairs(s2, npad), g2, be2, b2,
                           W3, _attn_mat(a_src3, 1), _attn_mat(a_dst3, 1))
  u2, s2 = _edge_pass(1, npad, ep, ht.reshape(2 * npad, HTW),
                      alt.reshape(2 * npad, 16), e2)
  return _final(npad, u2, _s_pairs(s2, npad), batch_r, g3, be3, b3,
                fc1_w, fc1_b, fc2_w, fc2_b)
